# Initial kernel scaffold; baseline (speedup 1.0000x reference)
import jax
import jax.numpy as jnp
from jax.experimental import pallas as pl


def kernel(x, pos, edge_index, batch, num_nodes, l, params):
    raise NotImplementedError("write your pallas kernel here")


# speedup vs baseline: 1.3700x; 1.3700x over previous
import functools

import jax
import jax.numpy as jnp
from jax.experimental import pallas as pl
from jax.experimental.pallas import tpu as pltpu

_N = 50000
_E = 800000
_G = 16
_H = 64
_BE = 4096
_EP = 802816
_BN = 6400
_NP = 51200


def _silu(a):
    return a * jax.nn.sigmoid(a)


def _enc_edge_body(gr_ref, gc_ref, w_ref, b1_ref, e2_ref, b2_ref, l_ref,
                   m_ref, g8_ref):
    lv = l_ref[0:1, 0:3]
    d = gr_ref[:, 0:3] - gc_ref[:, 0:3]
    d = jnp.where(d > 0.5 * lv, d - lv, d)
    d = jnp.where(d < -0.5 * lv, d + lv, d)
    r2 = jnp.sum(d * d, axis=-1, keepdims=True)
    xr = gr_ref[:, 3:4]
    xc = gc_ref[:, 3:4]
    pre = (xr * w_ref[0:1, :] + xc * w_ref[1:2, :] + r2 * w_ref[2:3, :]
           + b1_ref[0:1, :])
    m1 = _silu(pre)
    m = _silu(jnp.dot(m1, e2_ref[...], preferred_element_type=jnp.float32)
              + b2_ref[0:1, :])
    eidx = (pl.program_id(0) * _BE
            + jax.lax.broadcasted_iota(jnp.int32, (_BE, 1), 0))
    valid = (eidx < _E).astype(jnp.float32)
    m_ref[...] = m * valid
    dn = d / (jnp.sqrt(r2) + 1.0)
    lane = jax.lax.broadcasted_iota(jnp.int32, (_BE, 8), 1)
    g8 = jnp.where(lane < 3, jnp.pad(dn, ((0, 0), (0, 5))), 0.0)
    g8 = jnp.where(lane == 4, r2, g8)
    g8_ref[...] = g8 * valid


def _dec_edge_body(ar_ref, bc_ref, g8_ref, wr2_ref, b1_ref, e2_ref, b2_ref,
                   c1_ref, bc1_ref, c2t_ref, bc2_ref, out_ref):
    r2 = g8_ref[:, 4:5]
    pre = ar_ref[...] + bc_ref[...] + r2 * wr2_ref[0:1, :] + b1_ref[0:1, :]
    m = _silu(jnp.dot(_silu(pre), e2_ref[...],
                      preferred_element_type=jnp.float32) + b2_ref[0:1, :])
    c = _silu(jnp.dot(m, c1_ref[...], preferred_element_type=jnp.float32)
              + bc1_ref[0:1, :])
    cc = jnp.tanh(jnp.sum(c * c2t_ref[0:1, :], axis=-1, keepdims=True)
                  + bc2_ref[0:1, 0:1])
    lane = jax.lax.broadcasted_iota(jnp.int32, (_BE, 4), 1)
    out4 = g8_ref[:, 0:4] * cc + jnp.where(lane == 3, 1.0, 0.0)
    eidx = (pl.program_id(0) * _BE
            + jax.lax.broadcasted_iota(jnp.int32, (_BE, 1), 0))
    out_ref[...] = out4 * (eidx < _E).astype(jnp.float32)


def _node1_body(x_ref, aggm_ref, n1a_ref, n1b_ref, bn1_ref, n2_ref, bn2_ref,
                h_ref):
    pre = (x_ref[...] * n1a_ref[0:1, :]
           + jnp.dot(aggm_ref[...], n1b_ref[...],
                     preferred_element_type=jnp.float32)
           + bn1_ref[0:1, :])
    h_ref[...] = (jnp.dot(_silu(pre), n2_ref[...],
                          preferred_element_type=jnp.float32)
                  + bn2_ref[0:1, :])


def _node2_body(x_ref, zn_ref, emb0_ref, e1ab_ref, a_ref, b_ref):
    hd = x_ref[...] * emb0_ref[0:1, :] + zn_ref[...]
    ab = jnp.dot(hd, e1ab_ref[...], preferred_element_type=jnp.float32)
    a_ref[...] = ab[:, 0:_H]
    b_ref[...] = ab[:, _H:2 * _H]


def _row(v):
    return v.reshape(1, -1)


def kernel(x, pos, edge_index, batch, num_nodes, l, params):
    E = edge_index.shape[1]
    row = edge_index[0]
    col = edge_index[1]
    pad_e = _EP - E
    rowp = jnp.pad(row, (0, pad_e))
    colp = jnp.pad(col, (0, pad_e))

    p8 = jnp.concatenate(
        [pos, x[:, None], jnp.zeros((_N, 4), jnp.float32)], axis=1)
    gr = p8[rowp]
    gc = p8[colp]

    enc = params['enc']
    dec = params['dec']
    w_enc = jnp.pad(enc['e1'][0], ((0, 5), (0, 0)))
    l_row = jnp.pad(l, (0, 125)).reshape(1, 128)[:, :128]

    grid_e = _EP // _BE
    rep = lambda shape: pl.BlockSpec(shape, lambda i: (0, 0))
    m, g8 = pl.pallas_call(
        _enc_edge_body,
        grid=(grid_e,),
        in_specs=[
            pl.BlockSpec((_BE, 8), lambda i: (i, 0)),
            pl.BlockSpec((_BE, 8), lambda i: (i, 0)),
            rep((8, _H)),
            rep((1, _H)),
            rep((_H, _H)),
            rep((1, _H)),
            rep((1, 128)),
        ],
        out_specs=[
            pl.BlockSpec((_BE, _H), lambda i: (i, 0)),
            pl.BlockSpec((_BE, 8), lambda i: (i, 0)),
        ],
        out_shape=[
            jax.ShapeDtypeStruct((_EP, _H), jnp.float32),
            jax.ShapeDtypeStruct((_EP, 8), jnp.float32),
        ],
        compiler_params=pltpu.CompilerParams(
            dimension_semantics=("arbitrary",)),
    )(gr, gc, w_enc, _row(enc['e1'][1]), enc['e2'][0], _row(enc['e2'][1]),
      l_row)

    aggm = jax.ops.segment_sum(m, rowp, num_segments=_NP)

    xp = jnp.pad(x, (0, _NP - _N)).reshape(_NP, 1)
    n1 = enc['n1']
    grid_n = _NP // _BN
    h_enc = pl.pallas_call(
        _node1_body,
        grid=(grid_n,),
        in_specs=[
            pl.BlockSpec((_BN, 1), lambda i: (i, 0)),
            pl.BlockSpec((_BN, _H), lambda i: (i, 0)),
            rep((1, _H)),
            rep((_H, _H)),
            rep((1, _H)),
            rep((_H, _H)),
            rep((1, _H)),
        ],
        out_specs=pl.BlockSpec((_BN, _H), lambda i: (i, 0)),
        out_shape=jax.ShapeDtypeStruct((_NP, _H), jnp.float32),
        compiler_params=pltpu.CompilerParams(
            dimension_semantics=("arbitrary",)),
    )(xp, aggm, _row(n1[0][0]), n1[0][1:], _row(n1[1]), enc['n2'][0],
      _row(enc['n2'][1]))

    ssum = jax.ops.segment_sum(h_enc[:_N], batch, num_segments=_G,
                               indices_are_sorted=True)
    cntg = jax.ops.segment_sum(jnp.ones((_N, 1), jnp.float32), batch,
                               num_segments=_G, indices_are_sorted=True)
    hg = ssum / jnp.maximum(cntg, 1.0)
    mu = hg @ params['z_mu'][0] + params['z_mu'][1]
    sigma = jax.nn.softplus(hg @ params['z_sigma'][0] + params['z_sigma'][1])
    eps = jax.random.normal(jax.random.key(1), mu.shape, mu.dtype)
    z = mu + (sigma + 1e-6) * eps

    emb = params['emb']
    zemb = z @ emb[0][1:] + emb[1]
    batchp = jnp.pad(batch, (0, _NP - _N))
    znp = zemb[batchp]
    e1d = dec['e1'][0]
    e1ab = jnp.concatenate([e1d[:_H], e1d[_H:2 * _H]], axis=1)
    a_t, b_t = pl.pallas_call(
        _node2_body,
        grid=(grid_n,),
        in_specs=[
            pl.BlockSpec((_BN, 1), lambda i: (i, 0)),
            pl.BlockSpec((_BN, _H), lambda i: (i, 0)),
            rep((1, _H)),
            rep((_H, 2 * _H)),
        ],
        out_specs=[
            pl.BlockSpec((_BN, _H), lambda i: (i, 0)),
            pl.BlockSpec((_BN, _H), lambda i: (i, 0)),
        ],
        out_shape=[
            jax.ShapeDtypeStruct((_NP, _H), jnp.float32),
            jax.ShapeDtypeStruct((_NP, _H), jnp.float32),
        ],
        compiler_params=pltpu.CompilerParams(
            dimension_semantics=("arbitrary",)),
    )(xp, znp, _row(emb[0][0]), e1ab)

    ar = a_t[rowp]
    bc = b_t[colp]

    out4 = pl.pallas_call(
        _dec_edge_body,
        grid=(grid_e,),
        in_specs=[
            pl.BlockSpec((_BE, _H), lambda i: (i, 0)),
            pl.BlockSpec((_BE, _H), lambda i: (i, 0)),
            pl.BlockSpec((_BE, 8), lambda i: (i, 0)),
            rep((1, _H)),
            rep((1, _H)),
            rep((_H, _H)),
            rep((1, _H)),
            rep((_H, _H)),
            rep((1, _H)),
            rep((1, _H)),
            rep((1, 1)),
        ],
        out_specs=pl.BlockSpec((_BE, 4), lambda i: (i, 0)),
        out_shape=jax.ShapeDtypeStruct((_EP, 4), jnp.float32),
        compiler_params=pltpu.CompilerParams(
            dimension_semantics=("arbitrary",)),
    )(ar, bc, g8, _row(e1d[2 * _H]), _row(dec['e1'][1]), dec['e2'][0],
      _row(dec['e2'][1]), dec['c1'][0], _row(dec['c1'][1]),
      dec['c2'][0].reshape(1, _H), dec['c2'][1].reshape(1, 1))

    agg4 = jax.ops.segment_sum(out4, rowp, num_segments=_NP)[:_N]
    diff = agg4[:, 0:3] / jnp.maximum(agg4[:, 3:4], 1.0)
    diff = jnp.where(diff > 0.5 * l, diff - l, diff)
    diff = jnp.where(diff < -0.5 * l, diff + l, diff)
    return diff, z, mu, sigma


# speedup vs baseline: 3.2804x; 2.3945x over previous
import functools

import jax
import jax.numpy as jnp
from jax import lax
from jax.experimental import pallas as pl
from jax.experimental.pallas import tpu as pltpu
from jax.experimental.pallas import tpu_sc as plsc

_N = 50000
_E = 800000
_G = 16
_H = 64
_BE = 4096
_EP = 802816
_BN = 6400
_NP = 51200


_NC = 2
_NS = 16
_NW = _NC * _NS


def _silu(a):
    return a * jax.nn.sigmoid(a)


def _make_sc_gather2(ep, d, chunk):
    per_w = ep // _NW
    n_chunks = per_w // chunk
    assert per_w % chunk == 0 and per_w % 8 == 0
    mesh = plsc.VectorSubcoreMesh(core_axis_name="c", subcore_axis_name="s")

    def body(ta, tb, ia, ib, oa, ob, idx_v, rows_v, sem):
        wid = lax.axis_index("s") * _NC + lax.axis_index("c")
        base = wid * per_w

        @pl.loop(0, n_chunks)
        def _(ci):
            off = base + ci * chunk
            pltpu.sync_copy(ia.at[pl.ds(off, chunk)], idx_v)
            pltpu.async_copy(ta.at[idx_v], rows_v, sem).wait()
            pltpu.sync_copy(rows_v, oa.at[pl.ds(off, chunk)])
            pltpu.sync_copy(ib.at[pl.ds(off, chunk)], idx_v)
            pltpu.async_copy(tb.at[idx_v], rows_v, sem).wait()
            pltpu.sync_copy(rows_v, ob.at[pl.ds(off, chunk)])

    out_t = jax.ShapeDtypeStruct((ep, d), jnp.float32)
    return pl.kernel(
        body, mesh=mesh, out_type=[out_t, out_t],
        scratch_types=[
            pltpu.VMEM((chunk,), jnp.int32),
            pltpu.VMEM((chunk, d), jnp.float32),
            pltpu.SemaphoreType.DMA,
        ])


def _enc_edge_body(gr_ref, gc_ref, w_ref, b1_ref, e2_ref, b2_ref, l_ref,
                   m_ref, g8_ref):
    lv = l_ref[0:1, 0:3]
    d = gr_ref[:, 0:3] - gc_ref[:, 0:3]
    d = jnp.where(d > 0.5 * lv, d - lv, d)
    d = jnp.where(d < -0.5 * lv, d + lv, d)
    r2 = jnp.sum(d * d, axis=-1, keepdims=True)
    xr = gr_ref[:, 3:4]
    xc = gc_ref[:, 3:4]
    pre = (xr * w_ref[0:1, :] + xc * w_ref[1:2, :] + r2 * w_ref[2:3, :]
           + b1_ref[0:1, :])
    m1 = _silu(pre)
    m = _silu(jnp.dot(m1, e2_ref[...], preferred_element_type=jnp.float32)
              + b2_ref[0:1, :])
    eidx = (pl.program_id(0) * _BE
            + jax.lax.broadcasted_iota(jnp.int32, (_BE, 1), 0))
    valid = (eidx < _E).astype(jnp.float32)
    m_ref[...] = m * valid
    dn = d / (jnp.sqrt(r2) + 1.0)
    lane = jax.lax.broadcasted_iota(jnp.int32, (_BE, 8), 1)
    g8 = jnp.where(lane < 3, jnp.pad(dn, ((0, 0), (0, 5))), 0.0)
    g8 = jnp.where(lane == 4, r2, g8)
    g8_ref[...] = g8 * valid


def _dec_edge_body(ar_ref, bc_ref, g8_ref, wr2_ref, b1_ref, e2_ref, b2_ref,
                   c1_ref, bc1_ref, c2t_ref, bc2_ref, out_ref):
    r2 = g8_ref[:, 4:5]
    pre = (ar_ref[:, 0:_H] + bc_ref[:, _H:2 * _H] + r2 * wr2_ref[0:1, :]
           + b1_ref[0:1, :])
    m = _silu(jnp.dot(_silu(pre), e2_ref[...],
                      preferred_element_type=jnp.float32) + b2_ref[0:1, :])
    c = _silu(jnp.dot(m, c1_ref[...], preferred_element_type=jnp.float32)
              + bc1_ref[0:1, :])
    cc = jnp.tanh(jnp.sum(c * c2t_ref[0:1, :], axis=-1, keepdims=True)
                  + bc2_ref[0:1, 0:1])
    lane = jax.lax.broadcasted_iota(jnp.int32, (_BE, 4), 1)
    out4 = g8_ref[:, 0:4] * cc + jnp.where(lane == 3, 1.0, 0.0)
    eidx = (pl.program_id(0) * _BE
            + jax.lax.broadcasted_iota(jnp.int32, (_BE, 1), 0))
    out_ref[...] = out4 * (eidx < _E).astype(jnp.float32)


def _node1_body(x_ref, aggm_ref, n1a_ref, n1b_ref, bn1_ref, n2_ref, bn2_ref,
                h_ref):
    pre = (x_ref[...] * n1a_ref[0:1, :]
           + jnp.dot(aggm_ref[...], n1b_ref[...],
                     preferred_element_type=jnp.float32)
           + bn1_ref[0:1, :])
    h_ref[...] = (jnp.dot(_silu(pre), n2_ref[...],
                          preferred_element_type=jnp.float32)
                  + bn2_ref[0:1, :])


def _node2_body(x_ref, zn_ref, emb0_ref, e1ab_ref, ab_ref):
    hd = x_ref[...] * emb0_ref[0:1, :] + zn_ref[...]
    ab_ref[...] = jnp.dot(hd, e1ab_ref[...],
                          preferred_element_type=jnp.float32)


def _row(v):
    return v.reshape(1, -1)


def kernel(x, pos, edge_index, batch, num_nodes, l, params):
    E = edge_index.shape[1]
    row = edge_index[0]
    col = edge_index[1]
    pad_e = _EP - E
    rowp = jnp.pad(row, (0, pad_e))
    colp = jnp.pad(col, (0, pad_e))

    p128 = jnp.pad(jnp.concatenate([pos, x[:, None]], axis=1),
                   ((0, 0), (0, 124)))
    gr, gc = _make_sc_gather2(_EP, 128, 784)(p128, p128, rowp, colp)

    enc = params['enc']
    dec = params['dec']
    w_enc = jnp.pad(enc['e1'][0], ((0, 5), (0, 0)))
    l_row = jnp.pad(l, (0, 125)).reshape(1, 128)[:, :128]

    grid_e = _EP // _BE
    rep = lambda shape: pl.BlockSpec(shape, lambda i: (0, 0))
    m, g8 = pl.pallas_call(
        _enc_edge_body,
        grid=(grid_e,),
        in_specs=[
            pl.BlockSpec((_BE, 128), lambda i: (i, 0)),
            pl.BlockSpec((_BE, 128), lambda i: (i, 0)),
            rep((8, _H)),
            rep((1, _H)),
            rep((_H, _H)),
            rep((1, _H)),
            rep((1, 128)),
        ],
        out_specs=[
            pl.BlockSpec((_BE, _H), lambda i: (i, 0)),
            pl.BlockSpec((_BE, 8), lambda i: (i, 0)),
        ],
        out_shape=[
            jax.ShapeDtypeStruct((_EP, _H), jnp.float32),
            jax.ShapeDtypeStruct((_EP, 8), jnp.float32),
        ],
        compiler_params=pltpu.CompilerParams(
            dimension_semantics=("arbitrary",)),
    )(gr, gc, w_enc, _row(enc['e1'][1]), enc['e2'][0], _row(enc['e2'][1]),
      l_row)

    aggm = jax.ops.segment_sum(m, rowp, num_segments=_NP)

    xp = jnp.pad(x, (0, _NP - _N)).reshape(_NP, 1)
    n1 = enc['n1']
    grid_n = _NP // _BN
    h_enc = pl.pallas_call(
        _node1_body,
        grid=(grid_n,),
        in_specs=[
            pl.BlockSpec((_BN, 1), lambda i: (i, 0)),
            pl.BlockSpec((_BN, _H), lambda i: (i, 0)),
            rep((1, _H)),
            rep((_H, _H)),
            rep((1, _H)),
            rep((_H, _H)),
            rep((1, _H)),
        ],
        out_specs=pl.BlockSpec((_BN, _H), lambda i: (i, 0)),
        out_shape=jax.ShapeDtypeStruct((_NP, _H), jnp.float32),
        compiler_params=pltpu.CompilerParams(
            dimension_semantics=("arbitrary",)),
    )(xp, aggm, _row(n1[0][0]), n1[0][1:], _row(n1[1]), enc['n2'][0],
      _row(enc['n2'][1]))

    ssum = jax.ops.segment_sum(h_enc[:_N], batch, num_segments=_G,
                               indices_are_sorted=True)
    cntg = jax.ops.segment_sum(jnp.ones((_N, 1), jnp.float32), batch,
                               num_segments=_G, indices_are_sorted=True)
    hg = ssum / jnp.maximum(cntg, 1.0)
    mu = hg @ params['z_mu'][0] + params['z_mu'][1]
    sigma = jax.nn.softplus(hg @ params['z_sigma'][0] + params['z_sigma'][1])
    eps = jax.random.normal(jax.random.key(1), mu.shape, mu.dtype)
    z = mu + (sigma + 1e-6) * eps

    emb = params['emb']
    zemb = z @ emb[0][1:] + emb[1]
    batchp = jnp.pad(batch, (0, _NP - _N))
    znp = zemb[batchp]
    e1d = dec['e1'][0]
    e1ab = jnp.concatenate([e1d[:_H], e1d[_H:2 * _H]], axis=1)
    ab_t = pl.pallas_call(
        _node2_body,
        grid=(grid_n,),
        in_specs=[
            pl.BlockSpec((_BN, 1), lambda i: (i, 0)),
            pl.BlockSpec((_BN, _H), lambda i: (i, 0)),
            rep((1, _H)),
            rep((_H, 2 * _H)),
        ],
        out_specs=pl.BlockSpec((_BN, 2 * _H), lambda i: (i, 0)),
        out_shape=jax.ShapeDtypeStruct((_NP, 2 * _H), jnp.float32),
        compiler_params=pltpu.CompilerParams(
            dimension_semantics=("arbitrary",)),
    )(xp, znp, _row(emb[0][0]), e1ab)

    ar, bc = _make_sc_gather2(_EP, 2 * _H, 784)(ab_t, ab_t, rowp, colp)

    out4 = pl.pallas_call(
        _dec_edge_body,
        grid=(grid_e,),
        in_specs=[
            pl.BlockSpec((_BE, 2 * _H), lambda i: (i, 0)),
            pl.BlockSpec((_BE, 2 * _H), lambda i: (i, 0)),
            pl.BlockSpec((_BE, 8), lambda i: (i, 0)),
            rep((1, _H)),
            rep((1, _H)),
            rep((_H, _H)),
            rep((1, _H)),
            rep((_H, _H)),
            rep((1, _H)),
            rep((1, _H)),
            rep((1, 1)),
        ],
        out_specs=pl.BlockSpec((_BE, 4), lambda i: (i, 0)),
        out_shape=jax.ShapeDtypeStruct((_EP, 4), jnp.float32),
        compiler_params=pltpu.CompilerParams(
            dimension_semantics=("arbitrary",)),
    )(ar, bc, g8, _row(e1d[2 * _H]), _row(dec['e1'][1]), dec['e2'][0],
      _row(dec['e2'][1]), dec['c1'][0], _row(dec['c1'][1]),
      dec['c2'][0].reshape(1, _H), dec['c2'][1].reshape(1, 1))

    agg4 = jax.ops.segment_sum(out4, rowp, num_segments=_NP)[:_N]
    diff = agg4[:, 0:3] / jnp.maximum(agg4[:, 3:4], 1.0)
    diff = jnp.where(diff > 0.5 * l, diff - l, diff)
    diff = jnp.where(diff < -0.5 * l, diff + l, diff)
    return diff, z, mu, sigma


# speedup vs baseline: 4.8593x; 1.4813x over previous
import functools

import jax
import jax.numpy as jnp
from jax import lax
from jax.experimental import pallas as pl
from jax.experimental.pallas import tpu as pltpu
from jax.experimental.pallas import tpu_sc as plsc

_N = 50000
_E = 800000
_G = 16
_H = 64
_BE = 4096
_EP = 802816
_BN = 6400
_NP = 51200


_NC = 2
_NS = 16
_NW = _NC * _NS


def _silu(a):
    return a * jax.nn.sigmoid(a)


def _make_sc_gather2(ep, d, chunk):
    per_w = ep // _NW
    n_chunks = per_w // chunk
    assert per_w % chunk == 0 and per_w % 8 == 0
    mesh = plsc.VectorSubcoreMesh(core_axis_name="c", subcore_axis_name="s")

    def body(ta, tb, ia, ib, oa, ob, idx_v, rows_v, sem):
        wid = lax.axis_index("s") * _NC + lax.axis_index("c")
        base = wid * per_w

        @pl.loop(0, n_chunks)
        def _(ci):
            off = base + ci * chunk
            pltpu.sync_copy(ia.at[pl.ds(off, chunk)], idx_v)
            pltpu.async_copy(ta.at[idx_v], rows_v, sem).wait()
            pltpu.sync_copy(rows_v, oa.at[pl.ds(off, chunk)])
            pltpu.sync_copy(ib.at[pl.ds(off, chunk)], idx_v)
            pltpu.async_copy(tb.at[idx_v], rows_v, sem).wait()
            pltpu.sync_copy(rows_v, ob.at[pl.ds(off, chunk)])

    out_t = jax.ShapeDtypeStruct((ep, d), jnp.float32)
    return pl.kernel(
        body, mesh=mesh, out_type=[out_t, out_t],
        scratch_types=[
            pltpu.VMEM((chunk,), jnp.int32),
            pltpu.VMEM((chunk, d), jnp.float32),
            pltpu.SemaphoreType.DMA,
        ])


def _make_sc_scatter_lanes(ep, np_, chunk):
    per_sub = ep // _NS
    n_chunks = per_sub // chunk
    stripe = np_ // _NS
    assert per_sub % chunk == 0 and np_ % _NS == 0
    mesh = plsc.VectorSubcoreMesh(core_axis_name="c", subcore_axis_name="s")

    def body(m0, m1, idx_hbm, zeros_hbm, out_hbm, idx_v, vals_v, shared):
        c = lax.axis_index("c")
        s = lax.axis_index("s")
        pltpu.sync_copy(zeros_hbm.at[pl.ds(s * stripe, stripe)],
                        shared.at[pl.ds(s * stripe, stripe)])
        plsc.subcore_barrier()

        def run(mh):
            @pl.loop(0, n_chunks)
            def _(ci):
                off = s * per_sub + ci * chunk
                pltpu.sync_copy(idx_hbm.at[pl.ds(off, chunk)], idx_v)
                pltpu.sync_copy(mh.at[pl.ds(off, chunk)], vals_v)
                pltpu.sync_copy(vals_v, shared.at[idx_v], add=True)

        @pl.when(c == 0)
        def _():
            run(m0)

        @pl.when(c == 1)
        def _():
            run(m1)

        plsc.subcore_barrier()
        pltpu.sync_copy(shared.at[pl.ds(s * stripe, stripe)],
                        out_hbm.at[c, pl.ds(s * stripe, stripe)])

    return pl.kernel(
        body, mesh=mesh,
        out_type=jax.ShapeDtypeStruct((_NC, np_, 32), jnp.float32),
        scratch_types=[
            pltpu.VMEM((chunk,), jnp.int32),
            pltpu.VMEM((chunk, 32), jnp.float32),
            pltpu.VMEM_SHARED((np_, 32), jnp.float32),
        ],
        compiler_params=pltpu.CompilerParams(use_tc_tiling_on_sc=False))


def _make_sc_scatter_edges(ep, np_, chunk):
    per_sub = ep // _NW
    n_chunks = per_sub // chunk
    stripe = np_ // _NS
    assert per_sub % chunk == 0 and np_ % _NS == 0
    mesh = plsc.VectorSubcoreMesh(core_axis_name="c", subcore_axis_name="s")

    def body(t16, idx_hbm, zeros_hbm, out_hbm, idx_v, vals_v, shared):
        c = lax.axis_index("c")
        s = lax.axis_index("s")
        pltpu.sync_copy(zeros_hbm.at[pl.ds(s * stripe, stripe)],
                        shared.at[pl.ds(s * stripe, stripe)])
        plsc.subcore_barrier()
        base = (s * _NC + c) * per_sub

        @pl.loop(0, n_chunks)
        def _(ci):
            off = base + ci * chunk
            pltpu.sync_copy(idx_hbm.at[pl.ds(off, chunk)], idx_v)
            pltpu.sync_copy(t16.at[pl.ds(off, chunk)], vals_v)
            pltpu.sync_copy(vals_v, shared.at[idx_v], add=True)

        plsc.subcore_barrier()
        pltpu.sync_copy(shared.at[pl.ds(s * stripe, stripe)],
                        out_hbm.at[c, pl.ds(s * stripe, stripe)])

    return pl.kernel(
        body, mesh=mesh,
        out_type=jax.ShapeDtypeStruct((_NC, np_, 16), jnp.float32),
        scratch_types=[
            pltpu.VMEM((chunk,), jnp.int32),
            pltpu.VMEM((chunk, 16), jnp.float32),
            pltpu.VMEM_SHARED((np_, 16), jnp.float32),
        ],
        compiler_params=pltpu.CompilerParams(use_tc_tiling_on_sc=False))


def _enc_edge_body(gr_ref, gc_ref, w_ref, b1_ref, e2_ref, b2_ref, l_ref,
                   m0_ref, m1_ref, g8_ref):
    lv = l_ref[0:1, 0:3]
    d = gr_ref[:, 0:3] - gc_ref[:, 0:3]
    d = jnp.where(d > 0.5 * lv, d - lv, d)
    d = jnp.where(d < -0.5 * lv, d + lv, d)
    r2 = jnp.sum(d * d, axis=-1, keepdims=True)
    xr = gr_ref[:, 3:4]
    xc = gc_ref[:, 3:4]
    pre = (xr * w_ref[0:1, :] + xc * w_ref[1:2, :] + r2 * w_ref[2:3, :]
           + b1_ref[0:1, :])
    m1 = _silu(pre)
    m = _silu(jnp.dot(m1, e2_ref[...], preferred_element_type=jnp.float32)
              + b2_ref[0:1, :])
    eidx = (pl.program_id(0) * _BE
            + jax.lax.broadcasted_iota(jnp.int32, (_BE, 1), 0))
    valid = (eidx < _E).astype(jnp.float32)
    m = m * valid
    m0_ref[...] = m[:, 0:32]
    m1_ref[...] = m[:, 32:64]
    dn = d / (jnp.sqrt(r2) + 1.0)
    lane = jax.lax.broadcasted_iota(jnp.int32, (_BE, 8), 1)
    g8 = jnp.where(lane < 3, jnp.pad(dn, ((0, 0), (0, 5))), 0.0)
    g8 = jnp.where(lane == 4, r2, g8)
    g8_ref[...] = g8 * valid


def _dec_edge_body(ar_ref, bc_ref, g8_ref, wr2_ref, b1_ref, e2_ref, b2_ref,
                   c1_ref, bc1_ref, c2t_ref, bc2_ref, out_ref):
    r2 = g8_ref[:, 4:5]
    pre = (ar_ref[:, 0:_H] + bc_ref[:, _H:2 * _H] + r2 * wr2_ref[0:1, :]
           + b1_ref[0:1, :])
    m = _silu(jnp.dot(_silu(pre), e2_ref[...],
                      preferred_element_type=jnp.float32) + b2_ref[0:1, :])
    c = _silu(jnp.dot(m, c1_ref[...], preferred_element_type=jnp.float32)
              + bc1_ref[0:1, :])
    cc = jnp.tanh(jnp.sum(c * c2t_ref[0:1, :], axis=-1, keepdims=True)
                  + bc2_ref[0:1, 0:1])
    lane = jax.lax.broadcasted_iota(jnp.int32, (_BE, 4), 1)
    out4 = g8_ref[:, 0:4] * cc + jnp.where(lane == 3, 1.0, 0.0)
    eidx = (pl.program_id(0) * _BE
            + jax.lax.broadcasted_iota(jnp.int32, (_BE, 1), 0))
    out4 = out4 * (eidx < _E).astype(jnp.float32)
    out_ref[...] = jnp.pad(out4, ((0, 0), (0, 12)))


def _node1_body(x_ref, a0_ref, a1_ref, batch_ref, n1a_ref, n1b0_ref,
                n1b1_ref, bn1_ref, n2_ref, bn2_ref, hsum_ref):
    pre = (x_ref[...] * n1a_ref[0:1, :]
           + jnp.dot(a0_ref[0], n1b0_ref[...],
                     preferred_element_type=jnp.float32)
           + jnp.dot(a1_ref[0], n1b1_ref[...],
                     preferred_element_type=jnp.float32)
           + bn1_ref[0:1, :])
    h = (jnp.dot(_silu(pre), n2_ref[...], preferred_element_type=jnp.float32)
         + bn2_ref[0:1, :])
    nidx = (pl.program_id(0) * _BN
            + jax.lax.broadcasted_iota(jnp.int32, (_BN, 1), 0))
    valid = (nidx < _N).astype(jnp.float32)
    payload = jnp.pad(jnp.concatenate([h, valid], axis=1),
                      ((0, 0), (0, 63)))
    gid = jax.lax.broadcasted_iota(jnp.int32, (_BN, _G), 1)
    mask = (batch_ref[...] == gid).astype(jnp.float32) * valid
    part = jax.lax.dot_general(
        mask, payload, (((0,), (0,)), ((), ())),
        preferred_element_type=jnp.float32,
        precision=jax.lax.Precision.HIGHEST)

    @pl.when(pl.program_id(0) == 0)
    def _():
        hsum_ref[...] = jnp.zeros_like(hsum_ref)

    hsum_ref[...] += part


def _node2_body(x_ref, zn_ref, emb0_ref, e1ab_ref, ab_ref):
    hd = x_ref[...] * emb0_ref[0:1, :] + zn_ref[...]
    ab_ref[...] = jnp.dot(hd, e1ab_ref[...],
                          preferred_element_type=jnp.float32)


def _row(v):
    return v.reshape(1, -1)


def kernel(x, pos, edge_index, batch, num_nodes, l, params):
    E = edge_index.shape[1]
    row = edge_index[0]
    col = edge_index[1]
    pad_e = _EP - E
    rowp = jnp.pad(row, (0, pad_e))
    colp = jnp.pad(col, (0, pad_e))

    p128 = jnp.pad(jnp.concatenate([pos, x[:, None]], axis=1),
                   ((0, 0), (0, 124)))
    gr, gc = _make_sc_gather2(_EP, 128, 784)(p128, p128, rowp, colp)

    enc = params['enc']
    dec = params['dec']
    w_enc = jnp.pad(enc['e1'][0], ((0, 5), (0, 0)))
    l_row = jnp.pad(l, (0, 125)).reshape(1, 128)[:, :128]

    grid_e = _EP // _BE
    rep = lambda shape: pl.BlockSpec(shape, lambda i: (0, 0))
    m = pl.pallas_call(
        _enc_edge_body,
        grid=(grid_e,),
        in_specs=[
            pl.BlockSpec((_BE, 128), lambda i: (i, 0)),
            pl.BlockSpec((_BE, 128), lambda i: (i, 0)),
            rep((8, _H)),
            rep((1, _H)),
            rep((_H, _H)),
            rep((1, _H)),
            rep((1, 128)),
        ],
        out_specs=[
            pl.BlockSpec((_BE, 32), lambda i: (i, 0)),
            pl.BlockSpec((_BE, 32), lambda i: (i, 0)),
            pl.BlockSpec((_BE, 8), lambda i: (i, 0)),
        ],
        out_shape=[
            jax.ShapeDtypeStruct((_EP, 32), jnp.float32),
            jax.ShapeDtypeStruct((_EP, 32), jnp.float32),
            jax.ShapeDtypeStruct((_EP, 8), jnp.float32),
        ],
        compiler_params=pltpu.CompilerParams(
            dimension_semantics=("arbitrary",)),
    )(gr, gc, w_enc, _row(enc['e1'][1]), enc['e2'][0], _row(enc['e2'][1]),
      l_row)
    m0, m1, g8 = m

    zeros32 = jnp.zeros((_NP, 32), jnp.float32)
    aggm2 = _make_sc_scatter_lanes(_EP, _NP, 512)(m0, m1, rowp, zeros32)

    xp = jnp.pad(x, (0, _NP - _N)).reshape(_NP, 1)
    batchp2 = jnp.pad(batch, (0, _NP - _N)).reshape(_NP, 1)
    n1 = enc['n1']
    grid_n = _NP // _BN
    hsum = pl.pallas_call(
        _node1_body,
        grid=(grid_n,),
        in_specs=[
            pl.BlockSpec((_BN, 1), lambda i: (i, 0)),
            pl.BlockSpec((1, _BN, 32), lambda i: (0, i, 0)),
            pl.BlockSpec((1, _BN, 32), lambda i: (1, i, 0)),
            pl.BlockSpec((_BN, 1), lambda i: (i, 0)),
            rep((1, _H)),
            pl.BlockSpec((32, _H), lambda i: (0, 0)),
            pl.BlockSpec((32, _H), lambda i: (0, 0)),
            rep((1, _H)),
            rep((_H, _H)),
            rep((1, _H)),
        ],
        out_specs=pl.BlockSpec((_G, 128), lambda i: (0, 0)),
        out_shape=jax.ShapeDtypeStruct((_G, 128), jnp.float32),
        compiler_params=pltpu.CompilerParams(
            dimension_semantics=("arbitrary",)),
    )(xp, aggm2, aggm2, batchp2, _row(n1[0][0]), n1[0][1:33], n1[0][33:65],
      _row(n1[1]), enc['n2'][0], _row(enc['n2'][1]))

    hg = hsum[:, :_H] / jnp.maximum(hsum[:, _H:_H + 1], 1.0)
    mu = hg @ params['z_mu'][0] + params['z_mu'][1]
    sigma = jax.nn.softplus(hg @ params['z_sigma'][0] + params['z_sigma'][1])
    eps = jax.random.normal(jax.random.key(1), mu.shape, mu.dtype)
    z = mu + (sigma + 1e-6) * eps

    emb = params['emb']
    zemb = z @ emb[0][1:] + emb[1]
    batchp = jnp.pad(batch, (0, _NP - _N))
    znp = zemb[batchp]
    e1d = dec['e1'][0]
    e1ab = jnp.concatenate([e1d[:_H], e1d[_H:2 * _H]], axis=1)
    ab_t = pl.pallas_call(
        _node2_body,
        grid=(grid_n,),
        in_specs=[
            pl.BlockSpec((_BN, 1), lambda i: (i, 0)),
            pl.BlockSpec((_BN, _H), lambda i: (i, 0)),
            rep((1, _H)),
            rep((_H, 2 * _H)),
        ],
        out_specs=pl.BlockSpec((_BN, 2 * _H), lambda i: (i, 0)),
        out_shape=jax.ShapeDtypeStruct((_NP, 2 * _H), jnp.float32),
        compiler_params=pltpu.CompilerParams(
            dimension_semantics=("arbitrary",)),
    )(xp, znp, _row(emb[0][0]), e1ab)

    ar, bc = _make_sc_gather2(_EP, 2 * _H, 784)(ab_t, ab_t, rowp, colp)

    out4 = pl.pallas_call(
        _dec_edge_body,
        grid=(grid_e,),
        in_specs=[
            pl.BlockSpec((_BE, 2 * _H), lambda i: (i, 0)),
            pl.BlockSpec((_BE, 2 * _H), lambda i: (i, 0)),
            pl.BlockSpec((_BE, 8), lambda i: (i, 0)),
            rep((1, _H)),
            rep((1, _H)),
            rep((_H, _H)),
            rep((1, _H)),
            rep((_H, _H)),
            rep((1, _H)),
            rep((1, _H)),
            rep((1, 1)),
        ],
        out_specs=pl.BlockSpec((_BE, 16), lambda i: (i, 0)),
        out_shape=jax.ShapeDtypeStruct((_EP, 16), jnp.float32),
        compiler_params=pltpu.CompilerParams(
            dimension_semantics=("arbitrary",)),
    )(ar, bc, g8, _row(e1d[2 * _H]), _row(dec['e1'][1]), dec['e2'][0],
      _row(dec['e2'][1]), dec['c1'][0], _row(dec['c1'][1]),
      dec['c2'][0].reshape(1, _H), dec['c2'][1].reshape(1, 1))

    zeros16 = jnp.zeros((_NP, 16), jnp.float32)
    agg2 = _make_sc_scatter_edges(_EP, _NP, 896)(out4, rowp, zeros16)
    agg4 = (agg2[0] + agg2[1])[:_N]
    diff = agg4[:, 0:3] / jnp.maximum(agg4[:, 3:4], 1.0)
    diff = jnp.where(diff > 0.5 * l, diff - l, diff)
    diff = jnp.where(diff < -0.5 * l, diff + l, diff)
    return diff, z, mu, sigma
